# Initial kernel scaffold; baseline (speedup 1.0000x reference)
import jax
import jax.numpy as jnp
from jax.experimental import pallas as pl


def kernel(x, edge_index, batch, W_in, b_in, W1, b1, W2, b2, ln1_g, ln1_b, ln2_g, ln2_b):
    raise NotImplementedError("write your pallas kernel here")


# speedup vs baseline: 6.7753x; 6.7753x over previous
import functools

import jax
import jax.numpy as jnp
from jax import lax
from jax.experimental import pallas as pl
from jax.experimental.pallas import tpu as pltpu
from jax.experimental.pallas import tpu_sc as plsc

N = 10000
E = 160000
DIN = 256
H = 512
DOUT = 256
G = 8

NPAD = 10240
ROWS_PER_TILE = NPAD // 16
NB = 80
IB = 16
EPAD = 16 * NB * 128
RB = 2000
GRID = N // RB

@functools.lru_cache(maxsize=None)
def _mesh():
    return plsc.VectorSubcoreMesh(
        core_axis_name="c", subcore_axis_name="s", num_cores=2, num_subcores=16)


DROWS = NPAD // 128


def _sc_degree_body(dstidx_hbm, iden_hbm, zeros_hbm, out_hbm,
                    spmem, dstv, histv, idenv):
    cid = lax.axis_index("c")
    sid = lax.axis_index("s")
    pltpu.sync_copy(dstidx_hbm.at[sid], dstv)
    pltpu.sync_copy(iden_hbm, idenv)

    @pl.when(sid < DROWS // 8)
    def _():
        pltpu.sync_copy(zeros_hbm, spmem.at[pl.ds(sid * 8, 8)])
    zv = jnp.zeros((16,), jnp.float32)

    def zbody(r, _):
        for k in range(8):
            histv[r, pl.ds(k * 16, 16)] = zv
        return 0
    lax.fori_loop(0, DROWS, zbody, 0)
    plsc.subcore_barrier()

    ones = jnp.ones((16,), jnp.float32)

    def body(b, _):
        for k in range(8):
            idx = dstv[b, pl.ds(k * 16, 16)]
            plsc.addupdate_scatter(histv, [lax.div(idx, 128),
                                           lax.rem(idx, 128)], ones)
        return 0
    lax.fori_loop(0, NB, body, 0)
    pltpu.sync_copy(histv, spmem.at[idenv], add=True)
    plsc.subcore_barrier()

    @pl.when((cid == 0) & (sid < DROWS // 8))
    def _():
        pltpu.sync_copy(spmem.at[pl.ds(sid * 8, 8)],
                        out_hbm.at[pl.ds(sid * 8, 8)])


@functools.lru_cache(maxsize=None)
def _sc_degree():
    return pl.kernel(
        _sc_degree_body,
        out_type=jax.ShapeDtypeStruct((DROWS, 128), jnp.float32),
        mesh=_mesh(),
        compiler_params=pltpu.CompilerParams(needs_layout_passes=False),
        scratch_types=[
            pltpu.VMEM_SHARED((DROWS, 128), jnp.float32),
            pltpu.VMEM((NB, 128), jnp.int32),
            pltpu.VMEM((DROWS, 128), jnp.float32),
            pltpu.VMEM((DROWS,), jnp.int32),
        ],
    )


@functools.lru_cache(maxsize=None)
def _make_sc_agg(C):
    PC = C // 2

    @functools.partial(
        pl.kernel,
        out_type=jax.ShapeDtypeStruct((C, NPAD, 128), jnp.float32),
        mesh=_mesh(),
        scratch_types=[
            pltpu.VMEM_SHARED((NPAD, 128), jnp.float32),
            pltpu.VMEM((IB, 128), jnp.int32),
            pltpu.VMEM((IB, 128), jnp.int32),
            pltpu.VMEM((2, 128, 128), jnp.float32),
            pltpu.SemaphoreType.DMA,
        ],
    )
    def agg(u_hbm, srcidx_hbm, dstidx_hbm, zeros_hbm, out_hbm,
            spmem, srcv, dstv, rows, sem0):
        cid = lax.axis_index("c")
        sid = lax.axis_index("s")
        r0 = sid * ROWS_PER_TILE
        for j in range(PC):
            c = cid * PC + j
            pltpu.sync_copy(zeros_hbm.at[pl.ds(0, ROWS_PER_TILE)],
                            spmem.at[pl.ds(r0, ROWS_PER_TILE)])
            plsc.subcore_barrier()

            def group(g, _):
                pltpu.sync_copy(srcidx_hbm.at[c, sid, pl.ds(g * IB, IB)], srcv)
                pltpu.sync_copy(dstidx_hbm.at[sid, pl.ds(g * IB, IB)], dstv)

                def body(b, _):
                    pltpu.async_copy(u_hbm.at[srcv.at[b]], rows.at[0],
                                     sem0).wait()
                    pltpu.sync_copy(rows.at[0], spmem.at[dstv.at[b]], add=True)
                    return 0
                lax.fori_loop(0, IB, body, 0)
                return 0
            lax.fori_loop(0, NB // IB, group, 0)
            plsc.subcore_barrier()
            pltpu.sync_copy(spmem.at[pl.ds(r0, ROWS_PER_TILE)],
                            out_hbm.at[c].at[pl.ds(r0, ROWS_PER_TILE)])

    return agg


def _tc_a_body(x_ref, win_ref, bin_ref, w1_ref, deg_ref, out_ref):
    h0 = jnp.dot(x_ref[...], win_ref[...], preferred_element_type=jnp.float32)
    h0 = h0 + bin_ref[...]
    t1 = jnp.dot(h0, w1_ref[...], preferred_element_type=jnp.float32)
    dinv = lax.rsqrt(deg_ref[...] + 1.0)
    for c in range(H // 128):
        out_ref[c, :, :] = t1[:, c * 128:(c + 1) * 128] * dinv


def _ln_gelu(agg_ref, u_ref, dinv, bias_c, g_c, b_c, nchunks, width):
    h = (agg_ref[...] + u_ref[...]) * dinv[None, :, :] + bias_c[:, None, :]
    acc = jnp.zeros((RB, 1), jnp.float32)
    for c in range(nchunks):
        acc = acc + jnp.sum(h[c], axis=1, keepdims=True)
    mu = acc / float(width)
    vacc = jnp.zeros((RB, 1), jnp.float32)
    for c in range(nchunks):
        d = h[c] - mu
        vacc = vacc + jnp.sum(d * d, axis=1, keepdims=True)
    rstd = lax.rsqrt(vacc / float(width) + 1e-5)
    hn = (h - mu[None, :, :]) * rstd[None, :, :] * g_c[:, None, :] + b_c[:, None, :]
    return 0.5 * hn * (1.0 + lax.erf(hn * (2.0 ** -0.5)))


def _tc_d_body(agg_ref, u_ref, deg_ref, b1_ref, g1_ref, be1_ref, w2_ref, out_ref):
    dinv = lax.rsqrt(deg_ref[...] + 1.0)
    o1 = _ln_gelu(agg_ref, u_ref, dinv, b1_ref[...], g1_ref[...], be1_ref[...],
                  H // 128, H)
    t2 = jnp.zeros((RB, DOUT), jnp.float32)
    for c in range(H // 128):
        t2 = t2 + jnp.dot(o1[c], w2_ref[c, :, :], preferred_element_type=jnp.float32)
    for c in range(DOUT // 128):
        out_ref[c, :, :] = t2[:, c * 128:(c + 1) * 128] * dinv


def _tc_f_body(agg_ref, u_ref, deg_ref, bnd_ref, b2_ref, g2_ref, be2_ref,
               out_ref, sum_acc, max_acc):
    i = pl.program_id(0)
    dinv = lax.rsqrt(deg_ref[...] + 1.0)
    o2 = _ln_gelu(agg_ref, u_ref, dinv, b2_ref[...], g2_ref[...], be2_ref[...],
                  DOUT // 128, DOUT)
    hfull = jnp.concatenate([o2[c] for c in range(DOUT // 128)], axis=1)
    rowid = lax.broadcasted_iota(jnp.int32, (RB, DOUT), 0) + i * RB

    @pl.when(i == 0)
    def _():
        sum_acc[...] = jnp.zeros((G, DOUT), jnp.float32)
        max_acc[...] = jnp.full((G, DOUT), -jnp.inf, jnp.float32)

    for g in range(G):
        mask = (rowid >= bnd_ref[g]) & (rowid < bnd_ref[g + 1])
        sum_acc[g, :] = sum_acc[g, :] + jnp.sum(
            jnp.where(mask, hfull, 0.0), axis=0)
        max_acc[g, :] = jnp.maximum(
            max_acc[g, :], jnp.max(jnp.where(mask, hfull, -jnp.inf), axis=0))

    @pl.when(i == GRID - 1)
    def _():
        for g in range(G):
            cnt = (bnd_ref[g + 1] - bnd_ref[g]).astype(jnp.float32)
            out_ref[g, :] = (sum_acc[g, :] / jnp.maximum(cnt, 1.0)
                             + max_acc[g, :])


def _tc_a(x, W_in, b_in, W1, deg_bc):
    return pl.pallas_call(
        _tc_a_body,
        grid=(GRID,),
        in_specs=[
            pl.BlockSpec((RB, DIN), lambda i: (i, 0)),
            pl.BlockSpec((DIN, H), lambda i: (0, 0)),
            pl.BlockSpec((1, H), lambda i: (0, 0)),
            pl.BlockSpec((H, H), lambda i: (0, 0)),
            pl.BlockSpec((RB, 128), lambda i: (i, 0)),
        ],
        out_specs=pl.BlockSpec((H // 128, RB, 128), lambda i: (0, i, 0)),
        out_shape=jax.ShapeDtypeStruct((H // 128, N, 128), jnp.float32),
    )(x, W_in, b_in, W1, deg_bc)


def _tc_d(agg1, u1, deg_bc, b1c, g1c, be1c, W2r):
    return pl.pallas_call(
        _tc_d_body,
        grid=(GRID,),
        in_specs=[
            pl.BlockSpec((H // 128, RB, 128), lambda i: (0, i, 0)),
            pl.BlockSpec((H // 128, RB, 128), lambda i: (0, i, 0)),
            pl.BlockSpec((RB, 128), lambda i: (i, 0)),
            pl.BlockSpec((H // 128, 128), lambda i: (0, 0)),
            pl.BlockSpec((H // 128, 128), lambda i: (0, 0)),
            pl.BlockSpec((H // 128, 128), lambda i: (0, 0)),
            pl.BlockSpec((H // 128, 128, DOUT), lambda i: (0, 0, 0)),
        ],
        out_specs=pl.BlockSpec((DOUT // 128, RB, 128), lambda i: (0, i, 0)),
        out_shape=jax.ShapeDtypeStruct((DOUT // 128, N, 128), jnp.float32),
    )(agg1, u1, deg_bc, b1c, g1c, be1c, W2r)


def _tc_f(agg2, u2, deg_bc, bnd, b2c, g2c, be2c):
    return pl.pallas_call(
        _tc_f_body,
        grid=(GRID,),
        in_specs=[
            pl.BlockSpec((DOUT // 128, RB, 128), lambda i: (0, i, 0)),
            pl.BlockSpec((DOUT // 128, RB, 128), lambda i: (0, i, 0)),
            pl.BlockSpec((RB, 128), lambda i: (i, 0)),
            pl.BlockSpec(memory_space=pltpu.SMEM),
            pl.BlockSpec((DOUT // 128, 128), lambda i: (0, 0)),
            pl.BlockSpec((DOUT // 128, 128), lambda i: (0, 0)),
            pl.BlockSpec((DOUT // 128, 128), lambda i: (0, 0)),
        ],
        out_specs=pl.BlockSpec((G, DOUT), lambda i: (0, 0)),
        out_shape=jax.ShapeDtypeStruct((G, DOUT), jnp.float32),
        scratch_shapes=[
            pltpu.VMEM((G, DOUT), jnp.float32),
            pltpu.VMEM((G, DOUT), jnp.float32),
        ],
    )(agg2, u2, deg_bc, bnd, b2c, g2c, be2c)


def kernel(x, edge_index, batch, W_in, b_in, W1, b1, W2, b2,
           ln1_g, ln1_b, ln2_g, ln2_b):
    src = edge_index[0]
    dst = edge_index[1]
    pad = EPAD - E
    srcp = jnp.concatenate([src, jnp.zeros((pad,), jnp.int32)])
    dst_fill = N + (jnp.arange(pad, dtype=jnp.int32) % (NPAD - N))
    dstp = jnp.concatenate([dst, dst_fill])
    dst_tiles = dstp.reshape(16, NB, 128)
    src4 = (srcp[None, :] +
            (jnp.arange(4, dtype=jnp.int32) * N)[:, None]).reshape(4, 16, NB, 128)
    src2 = src4[:2]

    zeros_drows = jnp.zeros((8, 128), jnp.float32)
    iden = jnp.arange(DROWS, dtype=jnp.int32)
    zeros128 = jnp.zeros((ROWS_PER_TILE, 128), jnp.float32)
    bnd = jnp.searchsorted(batch, jnp.arange(G + 1, dtype=jnp.int32)
                           ).astype(jnp.int32)

    deg_tab = _sc_degree()(dst_tiles, iden, zeros_drows)
    deg_bc = jnp.broadcast_to(deg_tab.reshape(NPAD, 1)[:N], (N, 128))

    u1 = _tc_a(x, W_in, b_in.reshape(1, H), W1, deg_bc)
    agg1 = _make_sc_agg(4)(u1.reshape(4 * N, 128), src4, dst_tiles, zeros128)
    u2 = _tc_d(agg1, u1, deg_bc, b1.reshape(4, 128), ln1_g.reshape(4, 128),
               ln1_b.reshape(4, 128), W2.reshape(4, 128, DOUT))
    agg2 = _make_sc_agg(2)(u2.reshape(2 * N, 128), src2, dst_tiles, zeros128)
    return _tc_f(agg2, u2, deg_bc, bnd, b2.reshape(2, 128),
                 ln2_g.reshape(2, 128), ln2_b.reshape(2, 128))


# speedup vs baseline: 8.0244x; 1.1844x over previous
import functools

import jax
import jax.numpy as jnp
from jax import lax
from jax.experimental import pallas as pl
from jax.experimental.pallas import tpu as pltpu
from jax.experimental.pallas import tpu_sc as plsc

N = 10000
E = 160000
DIN = 256
H = 512
DOUT = 256
G = 8

NPAD = 10240
ROWS_PER_TILE = NPAD // 16
NB = 80
IB = 16
EPAD = 16 * NB * 128
RB = 2000
GRID = N // RB

@functools.lru_cache(maxsize=None)
def _mesh():
    return plsc.VectorSubcoreMesh(
        core_axis_name="c", subcore_axis_name="s", num_cores=2, num_subcores=16)


DROWS = NPAD // 128


def _sc_degree_body(dstidx_hbm, iden_hbm, zeros_hbm, out_hbm,
                    spmem, dstv, histv, idenv):
    cid = lax.axis_index("c")
    sid = lax.axis_index("s")
    pltpu.sync_copy(dstidx_hbm.at[sid], dstv)
    pltpu.sync_copy(iden_hbm, idenv)

    @pl.when(sid < DROWS // 8)
    def _():
        pltpu.sync_copy(zeros_hbm, spmem.at[pl.ds(sid * 8, 8)])
    zv = jnp.zeros((16,), jnp.float32)

    def zbody(r, _):
        for k in range(8):
            histv[r, pl.ds(k * 16, 16)] = zv
        return 0
    lax.fori_loop(0, DROWS, zbody, 0)
    plsc.subcore_barrier()

    ones = jnp.ones((16,), jnp.float32)

    def body(b, _):
        for k in range(8):
            idx = dstv[b, pl.ds(k * 16, 16)]
            plsc.addupdate_scatter(histv, [lax.div(idx, 128),
                                           lax.rem(idx, 128)], ones)
        return 0
    lax.fori_loop(0, NB, body, 0)
    pltpu.sync_copy(histv, spmem.at[idenv], add=True)
    plsc.subcore_barrier()

    @pl.when((cid == 0) & (sid < DROWS // 8))
    def _():
        pltpu.sync_copy(spmem.at[pl.ds(sid * 8, 8)],
                        out_hbm.at[pl.ds(sid * 8, 8)])


@functools.lru_cache(maxsize=None)
def _sc_degree():
    return pl.kernel(
        _sc_degree_body,
        out_type=jax.ShapeDtypeStruct((DROWS, 128), jnp.float32),
        mesh=_mesh(),
        compiler_params=pltpu.CompilerParams(needs_layout_passes=False),
        scratch_types=[
            pltpu.VMEM_SHARED((DROWS, 128), jnp.float32),
            pltpu.VMEM((NB, 128), jnp.int32),
            pltpu.VMEM((DROWS, 128), jnp.float32),
            pltpu.VMEM((DROWS,), jnp.int32),
        ],
    )


@functools.lru_cache(maxsize=None)
def _make_sc_agg(C):
    PC = C // 2

    @functools.partial(
        pl.kernel,
        out_type=jax.ShapeDtypeStruct((C, NPAD, 128), jnp.float32),
        mesh=_mesh(),
        scratch_types=[
            pltpu.VMEM_SHARED((NPAD, 128), jnp.float32),
            pltpu.VMEM((IB, 128), jnp.int32),
            pltpu.VMEM((IB, 128), jnp.int32),
            pltpu.VMEM((2, 128, 128), jnp.float32),
            pltpu.SemaphoreType.DMA,
            pltpu.SemaphoreType.DMA,
        ],
    )
    def agg(u_hbm, srcidx_hbm, dstidx_hbm, zeros_hbm, out_hbm,
            spmem, srcv, dstv, rows, sem0, sem1):
        cid = lax.axis_index("c")
        sid = lax.axis_index("s")
        r0 = sid * ROWS_PER_TILE
        for j in range(PC):
            c = cid * PC + j
            pltpu.sync_copy(zeros_hbm.at[pl.ds(0, ROWS_PER_TILE)],
                            spmem.at[pl.ds(r0, ROWS_PER_TILE)])
            plsc.subcore_barrier()

            def group(g, _):
                pltpu.sync_copy(srcidx_hbm.at[c, sid, pl.ds(g * IB, IB)], srcv)
                pltpu.sync_copy(dstidx_hbm.at[sid, pl.ds(g * IB, IB)], dstv)
                pltpu.async_copy(u_hbm.at[srcv.at[0]], rows.at[0], sem0)

                def body(p, _):
                    b0 = 2 * p
                    b1 = 2 * p + 1
                    pltpu.async_copy(u_hbm.at[srcv.at[b1]], rows.at[1], sem1)
                    pltpu.make_async_copy(u_hbm.at[srcv.at[b0]], rows.at[0],
                                          sem0).wait()
                    pltpu.sync_copy(rows.at[0], spmem.at[dstv.at[b0]],
                                    add=True)

                    @pl.when(p < IB // 2 - 1)
                    def _():
                        pltpu.async_copy(u_hbm.at[srcv.at[b1 + 1]],
                                         rows.at[0], sem0)
                    pltpu.make_async_copy(u_hbm.at[srcv.at[b1]], rows.at[1],
                                          sem1).wait()
                    pltpu.sync_copy(rows.at[1], spmem.at[dstv.at[b1]],
                                    add=True)
                    return 0
                lax.fori_loop(0, IB // 2, body, 0)
                return 0
            lax.fori_loop(0, NB // IB, group, 0)
            plsc.subcore_barrier()
            pltpu.sync_copy(spmem.at[pl.ds(r0, ROWS_PER_TILE)],
                            out_hbm.at[c].at[pl.ds(r0, ROWS_PER_TILE)])

    return agg


def _tc_a_body(x_ref, win_ref, bin_ref, w1_ref, deg_ref, out_ref):
    h0 = jnp.dot(x_ref[...], win_ref[...], preferred_element_type=jnp.float32)
    h0 = h0 + bin_ref[...]
    t1 = jnp.dot(h0, w1_ref[...], preferred_element_type=jnp.float32)
    dinv = lax.rsqrt(deg_ref[...] + 1.0)
    for c in range(H // 128):
        out_ref[c, :, :] = t1[:, c * 128:(c + 1) * 128] * dinv


def _ln_gelu(agg_ref, u_ref, dinv, bias_c, g_c, b_c, nchunks, width):
    h = (agg_ref[...] + u_ref[...]) * dinv[None, :, :] + bias_c[:, None, :]
    acc = jnp.zeros((RB, 1), jnp.float32)
    for c in range(nchunks):
        acc = acc + jnp.sum(h[c], axis=1, keepdims=True)
    mu = acc / float(width)
    vacc = jnp.zeros((RB, 1), jnp.float32)
    for c in range(nchunks):
        d = h[c] - mu
        vacc = vacc + jnp.sum(d * d, axis=1, keepdims=True)
    rstd = lax.rsqrt(vacc / float(width) + 1e-5)
    hn = (h - mu[None, :, :]) * rstd[None, :, :] * g_c[:, None, :] + b_c[:, None, :]
    return 0.5 * hn * (1.0 + lax.erf(hn * (2.0 ** -0.5)))


def _tc_d_body(agg_ref, u_ref, deg_ref, b1_ref, g1_ref, be1_ref, w2_ref, out_ref):
    dinv = lax.rsqrt(deg_ref[...] + 1.0)
    o1 = _ln_gelu(agg_ref, u_ref, dinv, b1_ref[...], g1_ref[...], be1_ref[...],
                  H // 128, H)
    t2 = jnp.zeros((RB, DOUT), jnp.float32)
    for c in range(H // 128):
        t2 = t2 + jnp.dot(o1[c], w2_ref[c, :, :], preferred_element_type=jnp.float32)
    for c in range(DOUT // 128):
        out_ref[c, :, :] = t2[:, c * 128:(c + 1) * 128] * dinv


def _tc_f_body(agg_ref, u_ref, deg_ref, bnd_ref, b2_ref, g2_ref, be2_ref,
               out_ref, sum_acc, max_acc):
    i = pl.program_id(0)
    dinv = lax.rsqrt(deg_ref[...] + 1.0)
    o2 = _ln_gelu(agg_ref, u_ref, dinv, b2_ref[...], g2_ref[...], be2_ref[...],
                  DOUT // 128, DOUT)
    hfull = jnp.concatenate([o2[c] for c in range(DOUT // 128)], axis=1)
    rowid = lax.broadcasted_iota(jnp.int32, (RB, DOUT), 0) + i * RB

    @pl.when(i == 0)
    def _():
        sum_acc[...] = jnp.zeros((G, DOUT), jnp.float32)
        max_acc[...] = jnp.full((G, DOUT), -jnp.inf, jnp.float32)

    for g in range(G):
        mask = (rowid >= bnd_ref[g]) & (rowid < bnd_ref[g + 1])
        sum_acc[g, :] = sum_acc[g, :] + jnp.sum(
            jnp.where(mask, hfull, 0.0), axis=0)
        max_acc[g, :] = jnp.maximum(
            max_acc[g, :], jnp.max(jnp.where(mask, hfull, -jnp.inf), axis=0))

    @pl.when(i == GRID - 1)
    def _():
        for g in range(G):
            cnt = (bnd_ref[g + 1] - bnd_ref[g]).astype(jnp.float32)
            out_ref[g, :] = (sum_acc[g, :] / jnp.maximum(cnt, 1.0)
                             + max_acc[g, :])


def _tc_a(x, W_in, b_in, W1, deg_bc):
    return pl.pallas_call(
        _tc_a_body,
        grid=(GRID,),
        in_specs=[
            pl.BlockSpec((RB, DIN), lambda i: (i, 0)),
            pl.BlockSpec((DIN, H), lambda i: (0, 0)),
            pl.BlockSpec((1, H), lambda i: (0, 0)),
            pl.BlockSpec((H, H), lambda i: (0, 0)),
            pl.BlockSpec((RB, 128), lambda i: (i, 0)),
        ],
        out_specs=pl.BlockSpec((H // 128, RB, 128), lambda i: (0, i, 0)),
        out_shape=jax.ShapeDtypeStruct((H // 128, N, 128), jnp.float32),
    )(x, W_in, b_in, W1, deg_bc)


def _tc_d(agg1, u1, deg_bc, b1c, g1c, be1c, W2r):
    return pl.pallas_call(
        _tc_d_body,
        grid=(GRID,),
        in_specs=[
            pl.BlockSpec((H // 128, RB, 128), lambda i: (0, i, 0)),
            pl.BlockSpec((H // 128, RB, 128), lambda i: (0, i, 0)),
            pl.BlockSpec((RB, 128), lambda i: (i, 0)),
            pl.BlockSpec((H // 128, 128), lambda i: (0, 0)),
            pl.BlockSpec((H // 128, 128), lambda i: (0, 0)),
            pl.BlockSpec((H // 128, 128), lambda i: (0, 0)),
            pl.BlockSpec((H // 128, 128, DOUT), lambda i: (0, 0, 0)),
        ],
        out_specs=pl.BlockSpec((DOUT // 128, RB, 128), lambda i: (0, i, 0)),
        out_shape=jax.ShapeDtypeStruct((DOUT // 128, N, 128), jnp.float32),
    )(agg1, u1, deg_bc, b1c, g1c, be1c, W2r)


def _tc_f(agg2, u2, deg_bc, bnd, b2c, g2c, be2c):
    return pl.pallas_call(
        _tc_f_body,
        grid=(GRID,),
        in_specs=[
            pl.BlockSpec((DOUT // 128, RB, 128), lambda i: (0, i, 0)),
            pl.BlockSpec((DOUT // 128, RB, 128), lambda i: (0, i, 0)),
            pl.BlockSpec((RB, 128), lambda i: (i, 0)),
            pl.BlockSpec(memory_space=pltpu.SMEM),
            pl.BlockSpec((DOUT // 128, 128), lambda i: (0, 0)),
            pl.BlockSpec((DOUT // 128, 128), lambda i: (0, 0)),
            pl.BlockSpec((DOUT // 128, 128), lambda i: (0, 0)),
        ],
        out_specs=pl.BlockSpec((G, DOUT), lambda i: (0, 0)),
        out_shape=jax.ShapeDtypeStruct((G, DOUT), jnp.float32),
        scratch_shapes=[
            pltpu.VMEM((G, DOUT), jnp.float32),
            pltpu.VMEM((G, DOUT), jnp.float32),
        ],
    )(agg2, u2, deg_bc, bnd, b2c, g2c, be2c)


def kernel(x, edge_index, batch, W_in, b_in, W1, b1, W2, b2,
           ln1_g, ln1_b, ln2_g, ln2_b):
    src = edge_index[0]
    dst = edge_index[1]
    pad = EPAD - E
    srcp = jnp.concatenate([src, jnp.zeros((pad,), jnp.int32)])
    dst_fill = N + (jnp.arange(pad, dtype=jnp.int32) % (NPAD - N))
    dstp = jnp.concatenate([dst, dst_fill])
    dst_tiles = dstp.reshape(16, NB, 128)
    src4 = (srcp[None, :] +
            (jnp.arange(4, dtype=jnp.int32) * N)[:, None]).reshape(4, 16, NB, 128)
    src2 = src4[:2]

    zeros_drows = jnp.zeros((8, 128), jnp.float32)
    iden = jnp.arange(DROWS, dtype=jnp.int32)
    zeros128 = jnp.zeros((ROWS_PER_TILE, 128), jnp.float32)
    bnd = jnp.searchsorted(batch, jnp.arange(G + 1, dtype=jnp.int32)
                           ).astype(jnp.int32)

    deg_tab = _sc_degree()(dst_tiles, iden, zeros_drows)
    deg_bc = jnp.broadcast_to(deg_tab.reshape(NPAD, 1)[:N], (N, 128))

    u1 = _tc_a(x, W_in, b_in.reshape(1, H), W1, deg_bc)
    agg1 = _make_sc_agg(4)(u1.reshape(4 * N, 128), src4, dst_tiles, zeros128)
    u2 = _tc_d(agg1, u1, deg_bc, b1.reshape(4, 128), ln1_g.reshape(4, 128),
               ln1_b.reshape(4, 128), W2.reshape(4, 128, DOUT))
    agg2 = _make_sc_agg(2)(u2.reshape(2 * N, 128), src2, dst_tiles, zeros128)
    return _tc_f(agg2, u2, deg_bc, bnd, b2.reshape(2, 128),
                 ln2_g.reshape(2, 128), ln2_b.reshape(2, 128))
